# Initial kernel scaffold; baseline (speedup 1.0000x reference)
import jax
import jax.numpy as jnp
from jax import lax
from jax.experimental import pallas as pl
from jax.experimental.pallas import tpu as pltpu
from jax.experimental.pallas import tpu_sc as plsc

_HEADS = 2
_OUT_CH = 2
_NB = 8
_GRADE_DIMS = (1, 3, 3, 1)

_N = 100000
_E = 1600000
_C = 2560
_G = _C // 128
_NW = 32
_NCHUNKS = _E // _C
_WIT = -(-_NCHUNKS // _NW)

_F32 = jnp.float32


def _proj_body(x_ref, m_ref, c_ref, z0_ref, z1_ref, t_ref):
    z32 = jnp.dot(x_ref[...], m_ref[...], preferred_element_type=_F32)
    t_ref[...] = jnp.dot(z32, c_ref[...], preferred_element_type=_F32)
    z0_ref[...] = z32[:, :16]
    z1_ref[...] = z32[:, 16:]


def _proj(x64, m, c32):
    bn = 2000
    grid = _N // bn
    return pl.pallas_call(
        _proj_body,
        grid=(grid,),
        in_specs=[
            pl.BlockSpec((bn, 64), lambda i: (i, 0)),
            pl.BlockSpec((64, 32), lambda i: (0, 0)),
            pl.BlockSpec((32, 4), lambda i: (0, 0)),
        ],
        out_specs=[
            pl.BlockSpec((bn, 16), lambda i: (i, 0)),
            pl.BlockSpec((bn, 16), lambda i: (i, 0)),
            pl.BlockSpec((bn, 4), lambda i: (i, 0)),
        ],
        out_shape=[
            jax.ShapeDtypeStruct((_N, 16), _F32),
            jax.ShapeDtypeStruct((_N, 16), _F32),
            jax.ShapeDtypeStruct((_N, 4), _F32),
        ],
    )(x64, m, c32)


def _scores_body(src_hbm, dst_hbm, t_hbm, zero2_hbm, expe_hbm, asum_hbm,
                 srcv, dstv, tsrc, tdst, ex0, ex1, inter, asum_sh, gsem, ssem):
    c = lax.axis_index("c")
    s = lax.axis_index("s")
    wid = s * 2 + c

    base = s * 6256

    @pl.when(s < 15)
    def _():
        pltpu.sync_copy(zero2_hbm.at[pl.ds(base, 6256)],
                        asum_sh.at[pl.ds(base, 6256)])

    @pl.when(s == 15)
    def _():
        pltpu.sync_copy(zero2_hbm.at[pl.ds(15 * 6256, 6160)],
                        asum_sh.at[pl.ds(15 * 6256, 6160)])

    plsc.subcore_barrier()

    lane = lax.iota(jnp.int32, 16)
    col0 = jnp.zeros((16,), jnp.int32)
    col1 = jnp.ones((16,), jnp.int32)
    col2 = jnp.full((16,), 2, jnp.int32)
    col3 = jnp.full((16,), 3, jnp.int32)

    def chunk(i, carry):
        cid = wid + _NW * i

        @pl.when(cid < _NCHUNKS)
        def _():
            pltpu.sync_copy(src_hbm.at[cid], srcv)
            pltpu.sync_copy(dst_hbm.at[cid], dstv)

            def fire(j, cc):
                pltpu.async_copy(t_hbm.at[srcv.at[j]],
                                 tsrc.at[pl.ds(j * 128, 128)], gsem)
                pltpu.async_copy(t_hbm.at[dstv.at[j]],
                                 tdst.at[pl.ds(j * 128, 128)], gsem)
                return cc

            lax.fori_loop(0, _G, fire, 0)
            pltpu.make_async_copy(t_hbm.at[pl.ds(0, _C)], tsrc, gsem).wait()
            pltpu.make_async_copy(t_hbm.at[pl.ds(0, _C)], tdst, gsem).wait()

            def grp(k, cc):
                b = k * 16
                ridx = b + lane
                ts0 = plsc.load_gather(tsrc, [ridx, col0])
                ts1 = plsc.load_gather(tsrc, [ridx, col1])
                td0 = plsc.load_gather(tdst, [ridx, col2])
                td1 = plsc.load_gather(tdst, [ridx, col3])
                e0 = ts0 + td0
                e1 = ts1 + td1
                x0 = jnp.exp(jnp.maximum(e0, 0.2 * e0))
                x1 = jnp.exp(jnp.maximum(e1, 0.2 * e1))
                ex0[pl.ds(b, 16)] = x0
                ex1[pl.ds(b, 16)] = x1
                plsc.store_scatter(inter, [ridx, col0], x0)
                plsc.store_scatter(inter, [ridx, col1], x1)
                return cc

            lax.fori_loop(0, _C // 16, grp, 0)
            pltpu.sync_copy(ex0, expe_hbm.at[0, cid])
            pltpu.sync_copy(ex1, expe_hbm.at[1, cid])

            def scat(j, cc):
                pltpu.async_copy(inter.at[pl.ds(j * 128, 128)],
                                 asum_sh.at[dstv.at[j]], ssem, add=True)
                return cc

            lax.fori_loop(0, _G, scat, 0)
            pltpu.make_async_copy(asum_hbm.at[0, pl.ds(0, _C)], inter,
                                  ssem).wait()

        return carry

    lax.fori_loop(0, _WIT, chunk, 0)
    plsc.subcore_barrier()

    @pl.when(s < 15)
    def _():
        pltpu.sync_copy(asum_sh.at[pl.ds(base, 6256)],
                        asum_hbm.at[c, pl.ds(base, 6256)])

    @pl.when(s == 15)
    def _():
        pltpu.sync_copy(asum_sh.at[pl.ds(15 * 6256, 6160)],
                        asum_hbm.at[c, pl.ds(15 * 6256, 6160)])


def _agg_body(src_hbm, dst_hbm, exh_hbm, zh_hbm, zero16_hbm, num_hbm,
              srcv, dstv, exv, zrows, out_sh, gsem, ssem):
    c = lax.axis_index("c")
    s = lax.axis_index("s")
    wid = s * 2 + c
    rbase = s * 6250

    pltpu.sync_copy(zero16_hbm.at[pl.ds(rbase, 6250)],
                    out_sh.at[pl.ds(rbase, 6250)])
    plsc.subcore_barrier()

    def chunk(i, carry):
        cid = wid + _NW * i

        @pl.when(cid < _NCHUNKS)
        def _():
            pltpu.sync_copy(src_hbm.at[cid], srcv)
            pltpu.sync_copy(dst_hbm.at[cid], dstv)
            pltpu.sync_copy(exh_hbm.at[cid], exv)

            def fire(j, cc):
                pltpu.async_copy(zh_hbm.at[srcv.at[j]],
                                 zrows.at[pl.ds(j * 128, 128)], gsem)
                return cc

            lax.fori_loop(0, _G, fire, 0)
            pltpu.make_async_copy(zh_hbm.at[pl.ds(0, _C)], zrows, gsem).wait()

            def grp(k, cc):
                b = k * 16
                for t in range(16):
                    a = plsc.load_gather(exv, [jnp.full((16,), b + t,
                                                        jnp.int32)])
                    zrows[b + t] = zrows[b + t] * a
                return cc

            lax.fori_loop(0, _C // 16, grp, 0)

            def scat(j, cc):
                pltpu.async_copy(zrows.at[pl.ds(j * 128, 128)],
                                 out_sh.at[dstv.at[j]], ssem, add=True)
                return cc

            lax.fori_loop(0, _G, scat, 0)
            pltpu.make_async_copy(zh_hbm.at[pl.ds(0, _C)], zrows, ssem).wait()

        return carry

    lax.fori_loop(0, _WIT, chunk, 0)
    plsc.subcore_barrier()
    pltpu.sync_copy(out_sh.at[pl.ds(rbase, 6250)],
                    num_hbm.at[c, pl.ds(rbase, 6250)])


def _sc_mesh():
    return plsc.VectorSubcoreMesh(core_axis_name="c", subcore_axis_name="s",
                                  num_cores=2, num_subcores=16)


def _scores_call(src3, dst3, t, zero2):
    return pl.kernel(
        _scores_body,
        out_type=(jax.ShapeDtypeStruct((2, _NCHUNKS, _C), _F32),
                  jax.ShapeDtypeStruct((2, _N, 2), _F32)),
        mesh=_sc_mesh(),
        scratch_types=[
            pltpu.VMEM((_G, 128), jnp.int32),
            pltpu.VMEM((_G, 128), jnp.int32),
            pltpu.VMEM((_C, 4), _F32),
            pltpu.VMEM((_C, 4), _F32),
            pltpu.VMEM((_C,), _F32),
            pltpu.VMEM((_C,), _F32),
            pltpu.VMEM((_C, 2), _F32),
            pltpu.VMEM_SHARED((_N, 2), _F32),
            pltpu.SemaphoreType.DMA,
            pltpu.SemaphoreType.DMA,
        ],
    )(src3, dst3, t, zero2)


def _agg_call(src3, dst3, exh, zh, zero16):
    return pl.kernel(
        _agg_body,
        out_type=jax.ShapeDtypeStruct((2, _N, 16), _F32),
        mesh=_sc_mesh(),
        scratch_types=[
            pltpu.VMEM((_G, 128), jnp.int32),
            pltpu.VMEM((_G, 128), jnp.int32),
            pltpu.VMEM((_C,), _F32),
            pltpu.VMEM((_C, 16), _F32),
            pltpu.VMEM_SHARED((_N, 16), _F32),
            pltpu.SemaphoreType.DMA,
            pltpu.SemaphoreType.DMA,
        ],
    )(src3, dst3, exh, zh, zero16)


def _merge_body(n0_ref, n1_ref, a_ref, o_ref):
    num0 = n0_ref[0] + n0_ref[1]
    num1 = n1_ref[0] + n1_ref[1]
    asum = a_ref[0] + a_ref[1]
    d0 = asum[:, 0:1] + 1e-16
    d1 = asum[:, 1:2] + 1e-16
    o_ref[...] = jnp.concatenate([num0 / d0, num1 / d1], axis=1)


def _merge(num0, num1, asum):
    bn = 2000
    grid = _N // bn
    return pl.pallas_call(
        _merge_body,
        grid=(grid,),
        in_specs=[
            pl.BlockSpec((2, bn, 16), lambda i: (0, i, 0)),
            pl.BlockSpec((2, bn, 16), lambda i: (0, i, 0)),
            pl.BlockSpec((2, bn, 2), lambda i: (0, i, 0)),
        ],
        out_specs=pl.BlockSpec((bn, 32), lambda i: (i, 0)),
        out_shape=jax.ShapeDtypeStruct((_N, 32), _F32),
    )(num0, num1, asum)


def _fold_att(a_list, w):
    parts = []
    for g, dg in enumerate(_GRADE_DIMS):
        parts.append(w[:, g][:, None, None]
                     * a_list[g].reshape(_HEADS, _OUT_CH, dg))
    att = jnp.concatenate(parts, axis=-1)
    eye2 = jnp.eye(2, dtype=_F32)
    return jnp.einsum('hob,hk->hobk', att, eye2).reshape(32, 2)


def kernel(x, edge_index, W, a_src_0, a_src_1, a_src_2, a_src_3,
           a_dst_0, a_dst_1, a_dst_2, a_dst_3, w_src, w_dst):
    x64 = x.reshape(_N, 64)
    bg = jnp.array([0, 1, 1, 1, 2, 2, 2, 3])
    wb = W[bg]
    eye8 = jnp.eye(8, dtype=_F32)
    m = jnp.einsum('boi,bc->iboc', wb, eye8).reshape(64, 32)
    c32 = jnp.concatenate(
        [_fold_att([a_src_0, a_src_1, a_src_2, a_src_3], w_src),
         _fold_att([a_dst_0, a_dst_1, a_dst_2, a_dst_3], w_dst)], axis=1)

    z0, z1, t = _proj(x64, m, c32)

    src3 = edge_index[0].reshape(_NCHUNKS, _G, 128)
    dst3 = edge_index[1].reshape(_NCHUNKS, _G, 128)
    zero2 = jnp.zeros((_N, 2), _F32)
    zero16 = jnp.zeros((_N, 16), _F32)

    expe, asum = _scores_call(src3, dst3, t, zero2)
    num0 = _agg_call(src3, dst3, expe[0], z0, zero16)
    num1 = _agg_call(src3, dst3, expe[1], z1, zero16)

    out32 = _merge(num0, num1, asum)
    return out32.reshape(_N, _HEADS * _OUT_CH, _NB)


# speedup vs baseline: 106.6066x; 106.6066x over previous
import jax
import jax.numpy as jnp
from jax import lax
from jax.experimental import pallas as pl
from jax.experimental.pallas import tpu as pltpu
from jax.experimental.pallas import tpu_sc as plsc

_HEADS = 2
_OUT_CH = 2
_NB = 8
_GRADE_DIMS = (1, 3, 3, 1)

_N = 100000
_E = 1600000
_C = 1280
_G = _C // 128
_NW = 32
_NCHUNKS = _E // _C
_WIT = -(-_NCHUNKS // _NW)

_F32 = jnp.float32


def _proj_body(x_ref, m_ref, c_ref, z0_ref, z1_ref, t_ref):
    z32 = jnp.dot(x_ref[...], m_ref[...], preferred_element_type=_F32)
    t_ref[...] = jnp.dot(z32, c_ref[...], preferred_element_type=_F32)
    z0_ref[...] = z32[:, :16]
    z1_ref[...] = z32[:, 16:]


def _proj(x64, m, c32):
    bn = 2000
    grid = _N // bn
    return pl.pallas_call(
        _proj_body,
        grid=(grid,),
        in_specs=[
            pl.BlockSpec((bn, 64), lambda i: (i, 0)),
            pl.BlockSpec((64, 32), lambda i: (0, 0)),
            pl.BlockSpec((32, 4), lambda i: (0, 0)),
        ],
        out_specs=[
            pl.BlockSpec((bn, 16), lambda i: (i, 0)),
            pl.BlockSpec((bn, 16), lambda i: (i, 0)),
            pl.BlockSpec((bn, 4), lambda i: (i, 0)),
        ],
        out_shape=[
            jax.ShapeDtypeStruct((_N, 16), _F32),
            jax.ShapeDtypeStruct((_N, 16), _F32),
            jax.ShapeDtypeStruct((_N, 4), _F32),
        ],
    )(x64, m, c32)


def _scores_body(src_hbm, dst_hbm, ts0_hbm, ts1_hbm, td0_hbm, td1_hbm,
                 zero1_hbm, expe_hbm, asum_hbm,
                 srcv, dstv, ts0, ts1, td0, td1, ex0, ex1,
                 as0_sh, as1_sh, gsem, ssem):
    c = lax.axis_index("c")
    s = lax.axis_index("s")
    wid = s * 2 + c

    base = s * 6256

    @pl.when(s < 15)
    def _():
        pltpu.sync_copy(zero1_hbm.at[pl.ds(base, 6256)],
                        as0_sh.at[pl.ds(base, 6256)])
        pltpu.sync_copy(zero1_hbm.at[pl.ds(base, 6256)],
                        as1_sh.at[pl.ds(base, 6256)])

    @pl.when(s == 15)
    def _():
        pltpu.sync_copy(zero1_hbm.at[pl.ds(15 * 6256, 6160)],
                        as0_sh.at[pl.ds(15 * 6256, 6160)])
        pltpu.sync_copy(zero1_hbm.at[pl.ds(15 * 6256, 6160)],
                        as1_sh.at[pl.ds(15 * 6256, 6160)])

    plsc.subcore_barrier()

    def chunk(i, carry):
        cid = wid + _NW * i

        @pl.when(cid < _NCHUNKS)
        def _():
            pltpu.sync_copy(src_hbm.at[cid], srcv)
            pltpu.sync_copy(dst_hbm.at[cid], dstv)

            def fire(j, cc):
                sl = pl.ds(j * 128, 128)
                pltpu.async_copy(ts0_hbm.at[srcv.at[j]], ts0.at[sl], gsem)
                pltpu.async_copy(ts1_hbm.at[srcv.at[j]], ts1.at[sl], gsem)
                pltpu.async_copy(td0_hbm.at[dstv.at[j]], td0.at[sl], gsem)
                pltpu.async_copy(td1_hbm.at[dstv.at[j]], td1.at[sl], gsem)
                return cc

            lax.fori_loop(0, _G, fire, 0)
            for _buf in (ts0, ts1, td0, td1):
                pltpu.make_async_copy(zero1_hbm.at[pl.ds(0, _C)], _buf,
                                      gsem).wait()

            def grp(k, cc):
                b = k * 16
                e0 = ts0[pl.ds(b, 16)] + td0[pl.ds(b, 16)]
                e1 = ts1[pl.ds(b, 16)] + td1[pl.ds(b, 16)]
                x0 = jnp.exp(jnp.maximum(e0, 0.2 * e0))
                x1 = jnp.exp(jnp.maximum(e1, 0.2 * e1))
                ex0[pl.ds(b, 16)] = x0
                ex1[pl.ds(b, 16)] = x1
                return cc

            lax.fori_loop(0, _C // 16, grp, 0)
            pltpu.sync_copy(ex0, expe_hbm.at[0, cid])
            pltpu.sync_copy(ex1, expe_hbm.at[1, cid])

            def scat(j, cc):
                sl = pl.ds(j * 128, 128)
                pltpu.async_copy(ex0.at[sl], as0_sh.at[dstv.at[j]], ssem,
                                 add=True)
                pltpu.async_copy(ex1.at[sl], as1_sh.at[dstv.at[j]], ssem,
                                 add=True)
                return cc

            lax.fori_loop(0, _G, scat, 0)
            pltpu.make_async_copy(zero1_hbm.at[pl.ds(0, _C)], ex0, ssem).wait()
            pltpu.make_async_copy(zero1_hbm.at[pl.ds(0, _C)], ex1, ssem).wait()

        return carry

    lax.fori_loop(0, _WIT, chunk, 0)
    plsc.subcore_barrier()

    @pl.when(s < 15)
    def _():
        pltpu.sync_copy(as0_sh.at[pl.ds(base, 6256)],
                        asum_hbm.at[c, 0, pl.ds(base, 6256)])
        pltpu.sync_copy(as1_sh.at[pl.ds(base, 6256)],
                        asum_hbm.at[c, 1, pl.ds(base, 6256)])

    @pl.when(s == 15)
    def _():
        pltpu.sync_copy(as0_sh.at[pl.ds(15 * 6256, 6160)],
                        asum_hbm.at[c, 0, pl.ds(15 * 6256, 6160)])
        pltpu.sync_copy(as1_sh.at[pl.ds(15 * 6256, 6160)],
                        asum_hbm.at[c, 1, pl.ds(15 * 6256, 6160)])


def _agg_body(src_hbm, dst_hbm, exh_hbm, zh_hbm, zero16_hbm, num_hbm,
              srcv, dstv, exv, zrows, out_sh, gsem, ssem):
    c = lax.axis_index("c")
    s = lax.axis_index("s")
    wid = s * 2 + c
    rbase = s * 6250

    pltpu.sync_copy(zero16_hbm.at[pl.ds(rbase, 6250)],
                    out_sh.at[pl.ds(rbase, 6250)])
    plsc.subcore_barrier()

    def chunk(i, carry):
        cid = wid + _NW * i

        @pl.when(cid < _NCHUNKS)
        def _():
            pltpu.sync_copy(src_hbm.at[cid], srcv)
            pltpu.sync_copy(dst_hbm.at[cid], dstv)
            pltpu.sync_copy(exh_hbm.at[cid], exv)

            def fire(j, cc):
                pltpu.async_copy(zh_hbm.at[srcv.at[j]],
                                 zrows.at[pl.ds(j * 128, 128)], gsem)
                return cc

            lax.fori_loop(0, _G, fire, 0)
            pltpu.make_async_copy(zh_hbm.at[pl.ds(0, _C)], zrows, gsem).wait()

            def grp(k, cc):
                b = k * 16
                for t in range(16):
                    a = plsc.load_gather(exv, [jnp.full((16,), b + t,
                                                        jnp.int32)])
                    zrows[b + t] = zrows[b + t] * a
                return cc

            lax.fori_loop(0, _C // 16, grp, 0)

            def scat(j, cc):
                pltpu.async_copy(zrows.at[pl.ds(j * 128, 128)],
                                 out_sh.at[dstv.at[j]], ssem, add=True)
                return cc

            lax.fori_loop(0, _G, scat, 0)
            pltpu.make_async_copy(zh_hbm.at[pl.ds(0, _C)], zrows, ssem).wait()

        return carry

    lax.fori_loop(0, _WIT, chunk, 0)
    plsc.subcore_barrier()
    pltpu.sync_copy(out_sh.at[pl.ds(rbase, 6250)],
                    num_hbm.at[c, pl.ds(rbase, 6250)])


def _sc_mesh():
    return plsc.VectorSubcoreMesh(core_axis_name="c", subcore_axis_name="s",
                                  num_cores=2, num_subcores=16)


def _scores_call(src3, dst3, ts0, ts1, td0, td1, zero1):
    return pl.kernel(
        _scores_body,
        out_type=(jax.ShapeDtypeStruct((2, _NCHUNKS, _C), _F32),
                  jax.ShapeDtypeStruct((2, 2, _N), _F32)),
        mesh=_sc_mesh(),
        compiler_params=pltpu.CompilerParams(use_tc_tiling_on_sc=False, needs_layout_passes=False),
        scratch_types=[
            pltpu.VMEM((_G, 128), jnp.int32),
            pltpu.VMEM((_G, 128), jnp.int32),
            pltpu.VMEM((_C,), _F32),
            pltpu.VMEM((_C,), _F32),
            pltpu.VMEM((_C,), _F32),
            pltpu.VMEM((_C,), _F32),
            pltpu.VMEM((_C,), _F32),
            pltpu.VMEM((_C,), _F32),
            pltpu.VMEM_SHARED((_N,), _F32),
            pltpu.VMEM_SHARED((_N,), _F32),
            pltpu.SemaphoreType.DMA,
            pltpu.SemaphoreType.DMA,
        ],
    )(src3, dst3, ts0, ts1, td0, td1, zero1)


def _agg_call(src3, dst3, exh, zh, zero16):
    return pl.kernel(
        _agg_body,
        out_type=jax.ShapeDtypeStruct((2, _N, 16), _F32),
        mesh=_sc_mesh(),
        compiler_params=pltpu.CompilerParams(use_tc_tiling_on_sc=False, needs_layout_passes=False),
        scratch_types=[
            pltpu.VMEM((_G, 128), jnp.int32),
            pltpu.VMEM((_G, 128), jnp.int32),
            pltpu.VMEM((_C,), _F32),
            pltpu.VMEM((_C, 16), _F32),
            pltpu.VMEM_SHARED((_N, 16), _F32),
            pltpu.SemaphoreType.DMA,
            pltpu.SemaphoreType.DMA,
        ],
    )(src3, dst3, exh, zh, zero16)


def _merge_body(n0_ref, n1_ref, a_ref, o_ref):
    num0 = n0_ref[0] + n0_ref[1]
    num1 = n1_ref[0] + n1_ref[1]
    d0 = (a_ref[:, 0, 0] + a_ref[:, 1, 0] + 1e-16)[:, None]
    d1 = (a_ref[:, 0, 1] + a_ref[:, 1, 1] + 1e-16)[:, None]
    o_ref[...] = jnp.concatenate([num0 / d0, num1 / d1], axis=1)


def _merge(num0, num1, asum):
    bn = 2000
    grid = _N // bn
    return pl.pallas_call(
        _merge_body,
        grid=(grid,),
        in_specs=[
            pl.BlockSpec((2, bn, 16), lambda i: (0, i, 0)),
            pl.BlockSpec((2, bn, 16), lambda i: (0, i, 0)),
            pl.BlockSpec((bn, 2, 2), lambda i: (i, 0, 0)),
        ],
        out_specs=pl.BlockSpec((bn, 32), lambda i: (i, 0)),
        out_shape=jax.ShapeDtypeStruct((_N, 32), _F32),
    )(num0, num1, asum)


def _fold_att(a_list, w):
    parts = []
    for g, dg in enumerate(_GRADE_DIMS):
        parts.append(w[:, g][:, None, None]
                     * a_list[g].reshape(_HEADS, _OUT_CH, dg))
    att = jnp.concatenate(parts, axis=-1)
    eye2 = jnp.eye(2, dtype=_F32)
    return jnp.einsum('hob,hk->hobk', att, eye2).reshape(32, 2)


def kernel(x, edge_index, W, a_src_0, a_src_1, a_src_2, a_src_3,
           a_dst_0, a_dst_1, a_dst_2, a_dst_3, w_src, w_dst):
    x64 = x.reshape(_N, 64)
    bg = jnp.array([0, 1, 1, 1, 2, 2, 2, 3])
    wb = W[bg]
    eye8 = jnp.eye(8, dtype=_F32)
    m = jnp.einsum('boi,bc->iboc', wb, eye8).reshape(64, 32)
    c32 = jnp.concatenate(
        [_fold_att([a_src_0, a_src_1, a_src_2, a_src_3], w_src),
         _fold_att([a_dst_0, a_dst_1, a_dst_2, a_dst_3], w_dst)], axis=1)

    z0, z1, t = _proj(x64, m, c32)

    src3 = edge_index[0].reshape(_NCHUNKS, _G, 128)
    dst3 = edge_index[1].reshape(_NCHUNKS, _G, 128)
    zero1 = jnp.zeros((_N,), _F32)
    zero16 = jnp.zeros((_N, 16), _F32)

    expe, asum = _scores_call(src3, dst3, t[:, 0], t[:, 1], t[:, 2], t[:, 3],
                              zero1)
    num0 = _agg_call(src3, dst3, expe[0], z0, zero16)
    num1 = _agg_call(src3, dst3, expe[1], z1, zero16)

    out32 = _merge(num0, num1, asum.transpose(2, 0, 1))
    return out32.reshape(_N, _HEADS * _OUT_CH, _NB)
